# baseline (speedup 1.0000x reference)
import functools

import jax
import jax.numpy as jnp
from jax import lax
from jax.experimental import pallas as pl
from jax.experimental.pallas import tpu as pltpu
from jax.experimental.pallas import tpu_sc as plsc

D_MODEL = 64
SCALE = float(D_MODEL) ** 0.5
NUM_CORES = 2
NUM_SUBCORES = 16
NUM_WORKERS = NUM_CORES * NUM_SUBCORES
CHUNK = 800


@functools.partial(jax.jit, static_argnums=(2,))
def _sc_embed(idx, table, n_rows):
    b_per_w = n_rows // NUM_WORKERS
    n_chunks = b_per_w // CHUNK
    mesh = plsc.VectorSubcoreMesh(core_axis_name="c", subcore_axis_name="s")

    @functools.partial(
        pl.kernel,
        mesh=mesh,
        out_type=jax.ShapeDtypeStruct((n_rows, D_MODEL), jnp.float32),
        scratch_types=[
            pltpu.VMEM((CHUNK,), jnp.int32),
            pltpu.VMEM((CHUNK, D_MODEL), jnp.float32),
            pltpu.SemaphoreType.DMA,
        ],
        compiler_params=pltpu.CompilerParams(use_tc_tiling_on_sc=False),
    )
    def k(idx_hbm, table_hbm, out_hbm, idx_v, rows_v, sem):
        wid = lax.axis_index("s") * NUM_CORES + lax.axis_index("c")
        base_w = wid * b_per_w

        def chunk_body(c, carry):
            base = base_w + c * CHUNK
            pltpu.sync_copy(idx_hbm.at[pl.ds(base, CHUNK)], idx_v)
            pltpu.async_copy(table_hbm.at[idx_v], rows_v, sem).wait()

            def row_body(r, carry2):
                for j in range(D_MODEL // 16):
                    sl = (r, pl.ds(j * 16, 16))
                    rows_v[sl] = rows_v[sl] * SCALE
                return carry2

            lax.fori_loop(0, CHUNK, row_body, 0)
            pltpu.sync_copy(rows_v, out_hbm.at[pl.ds(base, CHUNK)])
            return carry

        lax.fori_loop(0, n_chunks, chunk_body, 0)

    return k(idx, table)


def kernel(x, table):
    b, s = x.shape
    n = b * s
    xf = x.reshape(n).astype(jnp.int32)
    out = _sc_embed(xf, table, n)
    return out.reshape(b, s, D_MODEL)


# speedup vs baseline: 1.0450x; 1.0450x over previous
import functools

import jax
import jax.numpy as jnp
from jax import lax
from jax.experimental import pallas as pl
from jax.experimental.pallas import tpu as pltpu
from jax.experimental.pallas import tpu_sc as plsc

D_MODEL = 64
SCALE = float(D_MODEL) ** 0.5
NUM_CORES = 2
NUM_SUBCORES = 16
NUM_WORKERS = NUM_CORES * NUM_SUBCORES
CHUNK = 640
NBUF = 2
NIBUF = 3


@functools.partial(jax.jit, static_argnums=(2,))
def _sc_embed(idx, table, n_rows):
    b_per_w = n_rows // NUM_WORKERS
    n_chunks = b_per_w // CHUNK
    mesh = plsc.VectorSubcoreMesh(core_axis_name="c", subcore_axis_name="s")

    @functools.partial(
        pl.kernel,
        mesh=mesh,
        out_type=jax.ShapeDtypeStruct((n_rows, D_MODEL), jnp.float32),
        scratch_types=[
            [pltpu.VMEM((CHUNK,), jnp.int32) for _ in range(NIBUF)],
            [pltpu.VMEM((CHUNK, D_MODEL), jnp.float32) for _ in range(NBUF)],
            [pltpu.SemaphoreType.DMA for _ in range(NIBUF)],
            [pltpu.SemaphoreType.DMA for _ in range(NBUF)],
            [pltpu.SemaphoreType.DMA for _ in range(NBUF)],
        ],
        compiler_params=pltpu.CompilerParams(use_tc_tiling_on_sc=False),
    )
    def k(idx_hbm, table_hbm, out_hbm, idxs, rows, i_sems, g_sems, w_sems):
        wid = lax.axis_index("s") * NUM_CORES + lax.axis_index("c")
        base_w = wid * b_per_w

        def start_idx(c):
            b = c % NIBUF
            return pltpu.async_copy(
                idx_hbm.at[pl.ds(base_w + c * CHUNK, CHUNK)], idxs[b], i_sems[b]
            )

        def start_gather(c):
            return pltpu.async_copy(
                table_hbm.at[idxs[c % NIBUF]], rows[c % NBUF], g_sems[c % NBUF]
            )

        idx_copies = {c: start_idx(c) for c in range(min(2, n_chunks))}
        idx_copies.pop(0).wait()
        gathers = {0: start_gather(0)}
        writebacks = {}
        for c in range(n_chunks):
            gathers.pop(c).wait()
            if c + 1 < n_chunks:
                if c + 2 < n_chunks:
                    idx_copies[c + 2] = start_idx(c + 2)
                idx_copies.pop(c + 1).wait()
                if c + 1 >= NBUF:
                    writebacks.pop(c + 1 - NBUF).wait()
                gathers[c + 1] = start_gather(c + 1)

            buf = rows[c % NBUF]

            @plsc.parallel_loop(0, CHUNK, step=1, unroll=8)
            def _scale(r):
                for j in range(D_MODEL // 16):
                    sl = (r, pl.ds(j * 16, 16))
                    buf[sl] = buf[sl] * SCALE

            writebacks[c] = pltpu.async_copy(
                buf, out_hbm.at[pl.ds(base_w + c * CHUNK, CHUNK)], w_sems[c % NBUF]
            )
        for c in sorted(writebacks):
            writebacks.pop(c).wait()

    return k(idx, table)


def kernel(x, table):
    b, s = x.shape
    n = b * s
    xf = x.reshape(n).astype(jnp.int32)
    out = _sc_embed(xf, table, n)
    return out.reshape(b, s, D_MODEL)
